# baseline (speedup 1.0000x reference)
import functools

import jax
import jax.numpy as jnp
from jax import lax
from jax.experimental import pallas as pl
from jax.experimental.pallas import tpu as pltpu
from jax.experimental.pallas import tpu_sc as plsc

_USER_NUM = 1000000
_ITEM_NUM = 100000
_EMBED = 32
_BATCH = 4096
_N_ITEMS = 20
_L2_NORM = 1e-05

_NC = 2
_NS = 16
_NW = _NC * _NS
_BPW = _BATCH // _NW
_PPW = _BPW * _N_ITEMS
_NCHUNK = _PPW // 128


def _sc_body(users_hbm, items_hbm, bmap_hbm, utab_hbm, itab_hbm,
             pred_hbm, l2_hbm,
             uidx_v, iidx_v, bmap_v, urows_v, irows_v, pred_v, l2_v, sem):
    wid = lax.axis_index("s") * _NC + lax.axis_index("c")

    pltpu.sync_copy(users_hbm.at[wid], uidx_v)
    pltpu.sync_copy(items_hbm.at[wid], iidx_v)
    pltpu.sync_copy(bmap_hbm, bmap_v)

    copies = [pltpu.async_copy(utab_hbm.at[uidx_v], urows_v, sem)]
    for c in range(_NCHUNK):
        copies.append(
            pltpu.async_copy(itab_hbm.at[iidx_v.at[c]],
                             irows_v.at[pl.ds(c * 128, 128)], sem))
    for cp in copies:
        cp.wait()

    zeros = jnp.zeros((16,), jnp.float32)
    lane = lax.iota(jnp.int32, 16)

    def gbody(g, iacc):
        pvec = g * 16 + lane
        bvec = bmap_v[pl.ds(g * 16, 16)]
        acc = zeros
        for d in range(_EMBED):
            dvec = jnp.full((16,), d, jnp.int32)
            iv = plsc.load_gather(irows_v, [pvec, dvec])
            uv = plsc.load_gather(urows_v, [bvec, dvec])
            acc = acc + iv * uv
            iacc = iacc + iv * iv
        pred_v[pl.ds(g * 16, 16)] = acc
        return iacc

    iacc = lax.fori_loop(0, _PPW // 16, gbody, zeros)

    def ubody(b, uacc):
        u0 = urows_v[b, pl.ds(0, 16)]
        u1 = urows_v[b, pl.ds(16, 16)]
        return uacc + u0 * u0 + u1 * u1

    uacc = lax.fori_loop(0, _BPW, ubody, zeros)
    l2_v[...] = uacc * jnp.float32(_N_ITEMS) + iacc

    pltpu.sync_copy(pred_v, pred_hbm.at[pl.ds(wid * _PPW, _PPW)])
    pltpu.sync_copy(l2_v, l2_hbm.at[wid])


@jax.jit
def kernel(users, items, user_embedding, item_embedding):
    users_w = users.astype(jnp.int32).reshape(_NW, _BPW)
    items_w = items.astype(jnp.int32).reshape(_NW, _NCHUNK, 128)
    bmap = (jnp.arange(_PPW, dtype=jnp.int32) // _N_ITEMS)

    mesh = plsc.VectorSubcoreMesh(core_axis_name="c", subcore_axis_name="s")
    k = pl.kernel(
        _sc_body,
        out_type=(
            jax.ShapeDtypeStruct((_BATCH * _N_ITEMS,), jnp.float32),
            jax.ShapeDtypeStruct((_NW, 16), jnp.float32),
        ),
        mesh=mesh,
        compiler_params=pltpu.CompilerParams(use_tc_tiling_on_sc=False, needs_layout_passes=False),
        scratch_types=[
            pltpu.VMEM((_BPW,), jnp.int32),
            pltpu.VMEM((_NCHUNK, 128), jnp.int32),
            pltpu.VMEM((_PPW,), jnp.int32),
            pltpu.VMEM((_BPW, _EMBED), jnp.float32),
            pltpu.VMEM((_PPW, _EMBED), jnp.float32),
            pltpu.VMEM((_PPW,), jnp.float32),
            pltpu.VMEM((16,), jnp.float32),
            pltpu.SemaphoreType.DMA,
        ],
    )
    pred, l2_parts = k(users_w, items_w, bmap, user_embedding, item_embedding)
    l2 = jnp.float32(_L2_NORM) * jnp.sum(l2_parts)
    return pred.reshape(_BATCH, _N_ITEMS), l2


# speedup vs baseline: 1.1043x; 1.1043x over previous
import functools

import jax
import jax.numpy as jnp
from jax import lax
from jax.experimental import pallas as pl
from jax.experimental.pallas import tpu as pltpu
from jax.experimental.pallas import tpu_sc as plsc

_USER_NUM = 1000000
_ITEM_NUM = 100000
_EMBED = 32
_BATCH = 4096
_N_ITEMS = 20
_L2_NORM = 1e-05

_NC = 2
_NS = 16
_NW = _NC * _NS
_BPW = _BATCH // _NW
_PPW = _BPW * _N_ITEMS
_NCHUNK = _PPW // 128


def _sc_body(users_hbm, items_hbm, bmap_hbm, utab_hbm, itab_hbm,
             pred_hbm, l2_hbm,
             uidx_v, iidx_v, bmap_v, urows_v, irows_v, pred_v, l2_v, sem):
    wid = lax.axis_index("s") * _NC + lax.axis_index("c")

    pltpu.sync_copy(users_hbm.at[wid], uidx_v)
    pltpu.sync_copy(items_hbm.at[wid], iidx_v)
    pltpu.sync_copy(bmap_hbm, bmap_v)

    copies = [pltpu.async_copy(utab_hbm.at[uidx_v], urows_v, sem)]
    for c in range(_NCHUNK):
        copies.append(
            pltpu.async_copy(itab_hbm.at[iidx_v.at[c]],
                             irows_v.at[pl.ds(c * 128, 128)], sem))
    for cp in copies:
        cp.wait()

    zeros = jnp.zeros((16,), jnp.float32)
    lane = lax.iota(jnp.int32, 16)

    def bbody(b, carry):
        uacc, iacc = carry
        u0 = urows_v[b, pl.ds(0, 16)]
        u1 = urows_v[b, pl.ds(16, 16)]
        uacc = uacc + u0 * u0 + u1 * u1
        base = b * _N_ITEMS
        vec0 = zeros
        vec1 = zeros
        for l in range(_N_ITEMS):
            i0 = irows_v[base + l, pl.ds(0, 16)]
            i1 = irows_v[base + l, pl.ds(16, 16)]
            iacc = iacc + i0 * i0 + i1 * i1
            s = jnp.sum(i0 * u0 + i1 * u1)
            if l < 16:
                vec0 = jnp.where(lane == l, s, vec0)
            else:
                vec1 = jnp.where(lane == (l - 16), s, vec1)
        pred_v[pl.ds(b * 32, 16)] = vec0
        pred_v[pl.ds(b * 32 + 16, 16)] = vec1
        return (uacc, iacc)

    uacc, iacc = lax.fori_loop(0, _BPW, bbody, (zeros, zeros))
    l2_v[...] = uacc * jnp.float32(_N_ITEMS) + iacc

    pltpu.sync_copy(pred_v, pred_hbm.at[pl.ds(wid * _BPW * 32, _BPW * 32)])
    pltpu.sync_copy(l2_v, l2_hbm.at[wid])


@jax.jit
def kernel(users, items, user_embedding, item_embedding):
    users_w = users.astype(jnp.int32).reshape(_NW, _BPW)
    items_w = items.astype(jnp.int32).reshape(_NW, _NCHUNK, 128)
    bmap = (jnp.arange(_PPW, dtype=jnp.int32) // _N_ITEMS)

    mesh = plsc.VectorSubcoreMesh(core_axis_name="c", subcore_axis_name="s")
    k = pl.kernel(
        _sc_body,
        out_type=(
            jax.ShapeDtypeStruct((_BATCH * 32,), jnp.float32),
            jax.ShapeDtypeStruct((_NW, 16), jnp.float32),
        ),
        mesh=mesh,
        compiler_params=pltpu.CompilerParams(use_tc_tiling_on_sc=False, needs_layout_passes=False),
        scratch_types=[
            pltpu.VMEM((_BPW,), jnp.int32),
            pltpu.VMEM((_NCHUNK, 128), jnp.int32),
            pltpu.VMEM((_PPW,), jnp.int32),
            pltpu.VMEM((_BPW, _EMBED), jnp.float32),
            pltpu.VMEM((_PPW, _EMBED), jnp.float32),
            pltpu.VMEM((_BPW * 32,), jnp.float32),
            pltpu.VMEM((16,), jnp.float32),
            pltpu.SemaphoreType.DMA,
        ],
    )
    pred, l2_parts = k(users_w, items_w, bmap, user_embedding, item_embedding)
    l2 = jnp.float32(_L2_NORM) * jnp.sum(l2_parts)
    return pred.reshape(_BATCH, 32)[:, :_N_ITEMS], l2


# speedup vs baseline: 1.1079x; 1.0033x over previous
import jax
import jax.numpy as jnp
from jax import lax
from jax.experimental import pallas as pl
from jax.experimental.pallas import tpu as pltpu
from jax.experimental.pallas import tpu_sc as plsc

_USER_NUM = 1000000
_ITEM_NUM = 100000
_EMBED = 32
_BATCH = 4096
_N_ITEMS = 20
_L2_NORM = 1e-05

_NC = 2
_NS = 16
_NW = _NC * _NS
_BPW = _BATCH // _NW
_PPW = _BPW * _N_ITEMS
_NCHUNK = _PPW // 128


def _sc_body(users4_hbm, uoff_hbm, items_hbm, utab_hbm, itab_hbm,
             pred_hbm, l2_hbm,
             uidx_v, uoff_v, iidx_v, urows_v, irows_v, pred_v, l2_v, sem):
    wid = lax.axis_index("s") * _NC + lax.axis_index("c")

    pltpu.sync_copy(users4_hbm.at[wid], uidx_v)
    pltpu.sync_copy(uoff_hbm.at[wid], uoff_v)
    pltpu.sync_copy(items_hbm.at[wid], iidx_v)

    copies = [pltpu.async_copy(utab_hbm.at[uidx_v], urows_v, sem)]
    for c in range(_NCHUNK):
        copies.append(
            pltpu.async_copy(itab_hbm.at[iidx_v.at[c]],
                             irows_v.at[pl.ds(c * 128, 128)], sem))
    for cp in copies:
        cp.wait()

    zeros = jnp.zeros((16,), jnp.float32)
    lane = lax.iota(jnp.int32, 16)

    def bbody(b, carry):
        uacc, iacc = carry
        bvec = jnp.full((16,), 0, jnp.int32) + b
        ov = plsc.load_gather(uoff_v, [bvec])
        u0 = plsc.load_gather(urows_v, [bvec, ov + lane])
        u1 = plsc.load_gather(urows_v, [bvec, ov + lane + 16])
        uacc = uacc + u0 * u0 + u1 * u1
        base = b * _N_ITEMS
        vec0 = zeros
        vec1 = zeros
        for l in range(_N_ITEMS):
            i0 = irows_v[base + l, pl.ds(0, 16)]
            i1 = irows_v[base + l, pl.ds(16, 16)]
            iacc = iacc + i0 * i0 + i1 * i1
            s = jnp.sum(i0 * u0 + i1 * u1)
            if l < 16:
                vec0 = jnp.where(lane == l, s, vec0)
            else:
                vec1 = jnp.where(lane == (l - 16), s, vec1)
        pred_v[pl.ds(b * 32, 16)] = vec0
        pred_v[pl.ds(b * 32 + 16, 16)] = vec1
        return (uacc, iacc)

    uacc, iacc = lax.fori_loop(0, _BPW, bbody, (zeros, zeros))
    l2_v[...] = uacc * jnp.float32(_N_ITEMS) + iacc

    pltpu.sync_copy(pred_v, pred_hbm.at[pl.ds(wid * _BPW * 32, _BPW * 32)])
    pltpu.sync_copy(l2_v, l2_hbm.at[wid])


@jax.jit
def kernel(users, items, user_embedding, item_embedding):
    users_i = users.astype(jnp.int32).reshape(_NW, _BPW)
    users4_w = users_i // 4
    uoff_w = (users_i % 4) * _EMBED
    items_w = items.astype(jnp.int32).reshape(_NW, _NCHUNK, 128)
    utab4 = user_embedding.reshape(_USER_NUM // 4, 4 * _EMBED)

    mesh = plsc.VectorSubcoreMesh(core_axis_name="c", subcore_axis_name="s")
    k = pl.kernel(
        _sc_body,
        out_type=(
            jax.ShapeDtypeStruct((_BATCH * 32,), jnp.float32),
            jax.ShapeDtypeStruct((_NW, 16), jnp.float32),
        ),
        mesh=mesh,
        compiler_params=pltpu.CompilerParams(
            use_tc_tiling_on_sc=False, needs_layout_passes=False),
        scratch_types=[
            pltpu.VMEM((_BPW,), jnp.int32),
            pltpu.VMEM((_BPW,), jnp.int32),
            pltpu.VMEM((_NCHUNK, 128), jnp.int32),
            pltpu.VMEM((_BPW, 4 * _EMBED), jnp.float32),
            pltpu.VMEM((_PPW, _EMBED), jnp.float32),
            pltpu.VMEM((_BPW * 32,), jnp.float32),
            pltpu.VMEM((16,), jnp.float32),
            pltpu.SemaphoreType.DMA,
        ],
    )
    pred, l2_parts = k(users4_w, uoff_w, items_w, utab4, item_embedding)
    l2 = jnp.float32(_L2_NORM) * jnp.sum(l2_parts)
    return pred.reshape(_BATCH, 32)[:, :_N_ITEMS], l2


# speedup vs baseline: 6.2967x; 5.6832x over previous
import jax
import jax.numpy as jnp
from jax import lax
from jax.experimental import pallas as pl
from jax.experimental.pallas import tpu as pltpu
from jax.experimental.pallas import tpu_sc as plsc

_USER_NUM = 1000000
_ITEM_NUM = 100000
_EMBED = 32
_BATCH = 4096
_N_ITEMS = 20
_L2_NORM = 1e-05

_NC = 2
_NS = 16
_NW = _NC * _NS
_BPW = _BATCH // _NW
_PPW = _BPW * _N_ITEMS
_NCHUNK = _PPW // 128


def _sc_body(uro_hbm, items_hbm, itab_hbm,
             pred_hbm, l2_hbm,
             iidx_v, urows_v, irows_v, pred_v, l2_v, sem):
    wid = lax.axis_index("s") * _NC + lax.axis_index("c")

    pltpu.sync_copy(items_hbm.at[wid], iidx_v)
    pltpu.sync_copy(uro_hbm.at[wid], urows_v)

    copies = []
    for c in range(_NCHUNK):
        copies.append(
            pltpu.async_copy(itab_hbm.at[iidx_v.at[c]],
                             irows_v.at[pl.ds(c * 128, 128)], sem))
    for cp in copies:
        cp.wait()

    zeros = jnp.zeros((16,), jnp.float32)
    lane = lax.iota(jnp.int32, 16)

    def bbody(b, carry):
        uacc, iacc = carry
        u0 = urows_v[b, pl.ds(0, 16)]
        u1 = urows_v[b, pl.ds(16, 16)]
        uacc = uacc + u0 * u0 + u1 * u1
        base = b * _N_ITEMS
        vec0 = zeros
        vec1 = zeros
        for l in range(_N_ITEMS):
            i0 = irows_v[base + l, pl.ds(0, 16)]
            i1 = irows_v[base + l, pl.ds(16, 16)]
            iacc = iacc + i0 * i0 + i1 * i1
            s = jnp.sum(i0 * u0 + i1 * u1)
            if l < 16:
                vec0 = jnp.where(lane == l, s, vec0)
            else:
                vec1 = jnp.where(lane == (l - 16), s, vec1)
        pred_v[pl.ds(b * 32, 16)] = vec0
        pred_v[pl.ds(b * 32 + 16, 16)] = vec1
        return (uacc, iacc)

    uacc, iacc = lax.fori_loop(0, _BPW, bbody, (zeros, zeros))
    l2_v[...] = uacc * jnp.float32(_N_ITEMS) + iacc

    pltpu.sync_copy(pred_v, pred_hbm.at[pl.ds(wid * _BPW * 32, _BPW * 32)])
    pltpu.sync_copy(l2_v, l2_hbm.at[wid])


@jax.jit
def kernel(users, items, user_embedding, item_embedding):
    items_w = items.astype(jnp.int32).reshape(_NW, _NCHUNK, 128)
    uro = jnp.take(user_embedding, users.astype(jnp.int32)[:, 0], axis=0)
    uro_w = uro.reshape(_NW, _BPW, _EMBED)

    mesh = plsc.VectorSubcoreMesh(core_axis_name="c", subcore_axis_name="s")
    k = pl.kernel(
        _sc_body,
        out_type=(
            jax.ShapeDtypeStruct((_BATCH * 32,), jnp.float32),
            jax.ShapeDtypeStruct((_NW, 16), jnp.float32),
        ),
        mesh=mesh,
        compiler_params=pltpu.CompilerParams(
            use_tc_tiling_on_sc=False, needs_layout_passes=False),
        scratch_types=[
            pltpu.VMEM((_NCHUNK, 128), jnp.int32),
            pltpu.VMEM((_BPW, _EMBED), jnp.float32),
            pltpu.VMEM((_PPW, _EMBED), jnp.float32),
            pltpu.VMEM((_BPW * 32,), jnp.float32),
            pltpu.VMEM((16,), jnp.float32),
            pltpu.SemaphoreType.DMA,
        ],
    )
    pred, l2_parts = k(uro_w, items_w, item_embedding)
    l2 = jnp.float32(_L2_NORM) * jnp.sum(l2_parts)
    return pred.reshape(_BATCH, 32)[:, :_N_ITEMS], l2
